# baseline (speedup 1.0000x reference)
import functools

import numpy as np
import jax
import jax.numpy as jnp
from jax import lax
from jax.experimental import pallas as pl
from jax.experimental.pallas import tpu as pltpu, tpu_sc as plsc

B, C, D, GH, GW = 4, 12, 8, 16, 16
H = W = 512
NC, NS, L = 2, 16, 16
NW = NC * NS
ROWS_PER_W = (B * H) // NW
PV = W // L

_wgf = (np.arange(W, dtype=np.float32) / np.float32(W - 1)) * np.float32(2.0) - np.float32(1.0)
_ix = np.clip((_wgf + np.float32(1.0)) * np.float32(0.5) * np.float32(GW - 1),
              0.0, np.float32(GW - 1)).astype(np.float32)
_x0 = np.minimum(np.floor(_ix).astype(np.int32), GW - 2)
_TX = _ix - _x0.astype(np.float32)
_XB = (_x0 * D).astype(np.int32)


def _body(gt_hbm, fm_hbm, tx_hbm, xb_hbm, out_hbm,
          gridv, fmv, txv, xbv, lv, obuf, osem):
    wid = lax.axis_index("s") * NC + lax.axis_index("c")
    row0 = wid * ROWS_PER_W
    b = row0 // H
    h0 = row0 % H

    pltpu.sync_copy(tx_hbm, txv)
    pltpu.sync_copy(xb_hbm, xbv)
    pltpu.sync_copy(gt_hbm.at[b], gridv.at[pl.ds(0, C * GH * GW * D)])
    pltpu.sync_copy(fm_hbm.at[pl.ds(row0, ROWS_PER_W)], fmv)

    def row_body(r, carry):
        h = h0 + r
        buf = lax.rem(r, 2)
        @pl.when(r >= 2)
        def _():
            pltpu.make_async_copy(obuf.at[buf], out_hbm.at[b, :, h, :], osem).wait()
        hgf = h.astype(jnp.float32) * np.float32(1.0 / (H - 1)) * np.float32(2.0) - np.float32(1.0)
        iy = jnp.clip((hgf + np.float32(1.0)) * np.float32(0.5) * np.float32(GH - 1),
                      np.float32(0.0), np.float32(GH - 1))
        iyv = jnp.full((L,), iy, dtype=jnp.float32)
        y0v = jnp.minimum(iyv.astype(jnp.int32), GH - 2)
        wy1 = iyv - y0v.astype(jnp.float32)
        wy0 = jnp.full((L,), 1.0, dtype=jnp.float32) - wy1
        y0 = jnp.max(y0v)
        ybase = y0 * (GW * D)

        for c in range(C):
            for xv in range(GW * D // L):
                off = ybase + (c * (GH * GW * D) + xv * L)
                g0a = gridv[pl.ds(off, L)]
                g1a = gridv[pl.ds(off + GW * D, L)]
                g0b = gridv[pl.ds(off + 1, L)]
                g1b = gridv[pl.ds(off + GW * D + 1, L)]
                va = wy0 * g0a + wy1 * g1a
                vb = wy0 * g0b + wy1 * g1b
                pk = plsc.pack(va, vb, format=plsc.PackFormat.INTERLEAVED)
                lv[pl.ds(c * (GW * D) + xv * L, L)] = plsc.bitcast(pk, jnp.int32)

        @plsc.parallel_loop(0, PV, unroll=8)
        def pv_body(p):
            s = p * L
            fme = fmv[r, pl.ds(s, L)]
            txe = txv[pl.ds(s, L)]
            xbe = xbv[pl.ds(s, L)]
            iz = jnp.clip((fme + np.float32(1.0)) * np.float32(0.5 * (D - 1)),
                          np.float32(0.0), np.float32(D - 1))
            z0 = jnp.minimum(iz.astype(jnp.int32), D - 2)
            tz = iz - z0.astype(jnp.float32)
            i00 = xbe + z0
            i10 = i00 + D
            one = jnp.full((L,), 1.0, dtype=jnp.float32)
            uz = one - tz
            ux = one - txe
            w00 = ux * uz
            w01 = ux * tz
            w10 = txe * uz
            w11 = txe * tz
            for c in range(C):
                lc = lv.at[pl.ds(c * (GW * D), GW * D)]
                p0 = plsc.load_gather(lc, [i00])
                p1 = plsc.load_gather(lc, [i10])
                l0, l1 = plsc.unpack(plsc.bitcast(p0, jnp.bfloat16),
                                     format=plsc.PackFormat.INTERLEAVED,
                                     preferred_element_type=jnp.float32)
                r0, r1 = plsc.unpack(plsc.bitcast(p1, jnp.bfloat16),
                                     format=plsc.PackFormat.INTERLEAVED,
                                     preferred_element_type=jnp.float32)
                obuf[buf, c, pl.ds(s, L)] = (
                    (w00 * l0 + w01 * l1) + (w10 * r0 + w11 * r1))
        pltpu.make_async_copy(obuf.at[buf], out_hbm.at[b, :, h, :], osem).start()
        return carry

    lax.fori_loop(0, ROWS_PER_W, row_body, 0)
    for tail in (ROWS_PER_W - 2, ROWS_PER_W - 1):
        pltpu.make_async_copy(
            obuf.at[tail % 2], out_hbm.at[b, :, h0 + tail, :], osem).wait()


@jax.jit
def _slice_sc(gt, fm2, tx, xb):
    mesh = plsc.VectorSubcoreMesh(core_axis_name="c", subcore_axis_name="s")
    f = functools.partial(
        pl.kernel,
        out_type=jax.ShapeDtypeStruct((B, C, H, W), jnp.float32),
        mesh=mesh,
        compiler_params=pltpu.CompilerParams(needs_layout_passes=False),
        scratch_types=[
            pltpu.VMEM((C * GH * GW * D + L,), jnp.float32),
            pltpu.VMEM((ROWS_PER_W, W), jnp.float32),
            pltpu.VMEM((W,), jnp.float32),
            pltpu.VMEM((W,), jnp.int32),
            pltpu.VMEM((C * GW * D,), jnp.int32),
            pltpu.VMEM((2, C, W), jnp.float32),
            pltpu.SemaphoreType.DMA,
        ],
    )(_body)
    return f(gt, fm2, tx, xb)


def kernel(bilateral_grid, feature_map):
    gt = jnp.transpose(bilateral_grid, (0, 1, 3, 4, 2)).reshape(B, C * GH * GW * D)
    fm2 = feature_map.reshape(B * H, W)
    return _slice_sc(gt, fm2, jnp.asarray(_TX), jnp.asarray(_XB))


# speedup vs baseline: 1.1709x; 1.1709x over previous
import functools

import numpy as np
import jax
import jax.numpy as jnp
from jax import lax
from jax.experimental import pallas as pl
from jax.experimental.pallas import tpu as pltpu, tpu_sc as plsc

B, C, D, GH, GW = 4, 12, 8, 16, 16
H = W = 512
NC, NS, L = 2, 16, 16
NW = NC * NS
ROWS_PER_W = (B * H) // NW
PV = W // L

_wgf = (np.arange(W, dtype=np.float32) / np.float32(W - 1)) * np.float32(2.0) - np.float32(1.0)
_ix = np.clip((_wgf + np.float32(1.0)) * np.float32(0.5) * np.float32(GW - 1),
              0.0, np.float32(GW - 1)).astype(np.float32)
_x0 = np.minimum(np.floor(_ix).astype(np.int32), GW - 2)
_TX = _ix - _x0.astype(np.float32)
_XB = (_x0 * D).astype(np.int32)


def _body(gt_hbm, fm_hbm, tx_hbm, xb_hbm, out_hbm,
          gridv, fmv, txv, xbv, lv, obuf, osem):
    wid = lax.axis_index("s") * NC + lax.axis_index("c")
    row0 = wid * ROWS_PER_W
    b = row0 // H
    h0 = row0 % H

    pltpu.sync_copy(tx_hbm, txv)
    pltpu.sync_copy(xb_hbm, xbv)
    pltpu.sync_copy(gt_hbm.at[b], gridv.at[pl.ds(0, C * GH * GW * D)])
    pltpu.sync_copy(fm_hbm.at[pl.ds(row0, ROWS_PER_W)], fmv)

    def row_body(r, carry):
        h = h0 + r
        buf = lax.rem(r, 2)
        @pl.when(r >= 2)
        def _():
            pltpu.make_async_copy(obuf.at[buf], out_hbm.at[b, :, h, :], osem).wait()
        hgf = h.astype(jnp.float32) * np.float32(1.0 / (H - 1)) * np.float32(2.0) - np.float32(1.0)
        iy = jnp.clip((hgf + np.float32(1.0)) * np.float32(0.5) * np.float32(GH - 1),
                      np.float32(0.0), np.float32(GH - 1))
        iyv = jnp.full((L,), iy, dtype=jnp.float32)
        y0v = jnp.minimum(iyv.astype(jnp.int32), GH - 2)
        wy1 = iyv - y0v.astype(jnp.float32)
        wy0 = jnp.full((L,), 1.0, dtype=jnp.float32) - wy1
        y0 = jnp.max(y0v)
        ybase = y0 * (GW * D)

        for c in range(C):
            for xv in range(GW * D // L):
                off = ybase + (c * (GH * GW * D) + xv * L)
                g0a = gridv[pl.ds(off, L)]
                g1a = gridv[pl.ds(off + GW * D, L)]
                g0b = gridv[pl.ds(off + 1, L)]
                g1b = gridv[pl.ds(off + GW * D + 1, L)]
                va = wy0 * g0a + wy1 * g1a
                vb = wy0 * g0b + wy1 * g1b
                pk = plsc.pack(va, vb, format=plsc.PackFormat.INTERLEAVED)
                lv[pl.ds(c * (GW * D) + xv * L, L)] = plsc.bitcast(pk, jnp.int32)

        @plsc.parallel_loop(0, PV, unroll=8)
        def pv_body(p):
            s = p * L
            fme = fmv[r, pl.ds(s, L)]
            txe = txv[pl.ds(s, L)]
            xbe = xbv[pl.ds(s, L)]
            iz = jnp.clip((fme + np.float32(1.0)) * np.float32(0.5 * (D - 1)),
                          np.float32(0.0), np.float32(D - 1))
            z0 = jnp.minimum(iz.astype(jnp.int32), D - 2)
            tz = iz - z0.astype(jnp.float32)
            i00 = xbe + z0
            i10 = i00 + D
            one = jnp.full((L,), 1.0, dtype=jnp.float32)
            uz = one - tz
            ux = one - txe
            w00 = ux * uz
            w01 = ux * tz
            w10 = txe * uz
            w11 = txe * tz
            pw0 = plsc.pack(w00, w01, format=plsc.PackFormat.INTERLEAVED)
            pw1 = plsc.pack(w10, w11, format=plsc.PackFormat.INTERLEAVED)
            for c in range(C):
                lc = lv.at[pl.ds(c * (GW * D), GW * D)]
                p0 = plsc.load_gather(lc, [i00])
                p1 = plsc.load_gather(lc, [i10])
                t = (pw0 * plsc.bitcast(p0, jnp.bfloat16)
                     + pw1 * plsc.bitcast(p1, jnp.bfloat16))
                ta, tb = plsc.unpack(t, format=plsc.PackFormat.INTERLEAVED,
                                     preferred_element_type=jnp.float32)
                obuf[buf, c, pl.ds(s, L)] = ta + tb
        pltpu.make_async_copy(obuf.at[buf], out_hbm.at[b, :, h, :], osem).start()
        return carry

    lax.fori_loop(0, ROWS_PER_W, row_body, 0)
    for tail in (ROWS_PER_W - 2, ROWS_PER_W - 1):
        pltpu.make_async_copy(
            obuf.at[tail % 2], out_hbm.at[b, :, h0 + tail, :], osem).wait()


@jax.jit
def _slice_sc(gt, fm2, tx, xb):
    mesh = plsc.VectorSubcoreMesh(core_axis_name="c", subcore_axis_name="s")
    f = functools.partial(
        pl.kernel,
        out_type=jax.ShapeDtypeStruct((B, C, H, W), jnp.float32),
        mesh=mesh,
        compiler_params=pltpu.CompilerParams(needs_layout_passes=False),
        scratch_types=[
            pltpu.VMEM((C * GH * GW * D + L,), jnp.float32),
            pltpu.VMEM((ROWS_PER_W, W), jnp.float32),
            pltpu.VMEM((W,), jnp.float32),
            pltpu.VMEM((W,), jnp.int32),
            pltpu.VMEM((C * GW * D,), jnp.int32),
            pltpu.VMEM((2, C, W), jnp.float32),
            pltpu.SemaphoreType.DMA,
        ],
    )(_body)
    return f(gt, fm2, tx, xb)


def kernel(bilateral_grid, feature_map):
    gt = jnp.transpose(bilateral_grid, (0, 1, 3, 4, 2)).reshape(B, C * GH * GW * D)
    fm2 = feature_map.reshape(B * H, W)
    return _slice_sc(gt, fm2, jnp.asarray(_TX), jnp.asarray(_XB))


# speedup vs baseline: 1.5690x; 1.3400x over previous
import functools

import numpy as np
import jax
import jax.numpy as jnp
from jax import lax
from jax.experimental import pallas as pl
from jax.experimental.pallas import tpu as pltpu, tpu_sc as plsc

B, C, D, GH, GW = 4, 12, 8, 16, 16
H = W = 512
NC, NS, L = 2, 16, 16
NW = NC * NS
ROWS_PER_W = (B * H) // NW
PV = W // L

_wgf = (np.arange(W, dtype=np.float32) / np.float32(W - 1)) * np.float32(2.0) - np.float32(1.0)
_ix = np.clip((_wgf + np.float32(1.0)) * np.float32(0.5) * np.float32(GW - 1),
              0.0, np.float32(GW - 1)).astype(np.float32)
_x0 = np.minimum(np.floor(_ix).astype(np.int32), GW - 2)
_TX = _ix - _x0.astype(np.float32)
_XB = (_x0 * D).astype(np.int32)


def _body(gt_hbm, fm_hbm, tx_hbm, xb_hbm, out_hbm,
          gridv, fmv, txv, xbv, lv, obuf, osem):
    wid = lax.axis_index("s") * NC + lax.axis_index("c")
    row0 = wid * ROWS_PER_W
    b = row0 // H
    h0 = row0 % H

    pltpu.sync_copy(tx_hbm, txv)
    pltpu.sync_copy(xb_hbm, xbv)
    pltpu.sync_copy(gt_hbm.at[b], gridv.at[pl.ds(0, C * GH * GW * D)])
    pltpu.sync_copy(fm_hbm.at[pl.ds(row0, ROWS_PER_W)], fmv)

    def row_body(r, carry):
        h = h0 + r
        buf = lax.rem(r, 2)
        @pl.when(r >= 2)
        def _():
            pltpu.make_async_copy(obuf.at[buf], out_hbm.at[b, :, h, :], osem).wait()
        hgf = h.astype(jnp.float32) * np.float32(1.0 / (H - 1)) * np.float32(2.0) - np.float32(1.0)
        iy = jnp.clip((hgf + np.float32(1.0)) * np.float32(0.5) * np.float32(GH - 1),
                      np.float32(0.0), np.float32(GH - 1))
        iyv = jnp.full((L,), iy, dtype=jnp.float32)
        y0v = jnp.minimum(iyv.astype(jnp.int32), GH - 2)
        wy1 = iyv - y0v.astype(jnp.float32)
        wy0 = jnp.full((L,), 1.0, dtype=jnp.float32) - wy1
        y0 = jnp.max(y0v)
        ybase = y0 * (GW * D)

        @plsc.parallel_loop(0, C * (GW * D // L), unroll=8)
        def build(g):
            off = ybase + g * L + lax.shift_right_logical(g, 3) * (GH * GW * D - GW * D)
            g0a = gridv[pl.ds(off, L)]
            g1a = gridv[pl.ds(off + GW * D, L)]
            g0b = gridv[pl.ds(off + 1, L)]
            g1b = gridv[pl.ds(off + GW * D + 1, L)]
            va = wy0 * g0a + wy1 * g1a
            vb = wy0 * g0b + wy1 * g1b
            pk = plsc.pack(va, vb, format=plsc.PackFormat.INTERLEAVED)
            lv[pl.ds(g * L, L)] = plsc.bitcast(pk, jnp.int32)

        @plsc.parallel_loop(0, PV, unroll=8)
        def pv_body(p):
            s = p * L
            fme = fmv[r, pl.ds(s, L)]
            txe = txv[pl.ds(s, L)]
            xbe = xbv[pl.ds(s, L)]
            iz = jnp.clip((fme + np.float32(1.0)) * np.float32(0.5 * (D - 1)),
                          np.float32(0.0), np.float32(D - 1))
            z0 = jnp.minimum(iz.astype(jnp.int32), D - 2)
            tz = iz - z0.astype(jnp.float32)
            i00 = xbe + z0
            i10 = i00 + D
            one = jnp.full((L,), 1.0, dtype=jnp.float32)
            uz = one - tz
            ux = one - txe
            w00 = ux * uz
            w01 = ux * tz
            w10 = txe * uz
            w11 = txe * tz
            pw0 = plsc.pack(w00, w01, format=plsc.PackFormat.INTERLEAVED)
            pw1 = plsc.pack(w10, w11, format=plsc.PackFormat.INTERLEAVED)
            for c in range(C):
                lc = lv.at[pl.ds(c * (GW * D), GW * D)]
                p0 = plsc.load_gather(lc, [i00])
                p1 = plsc.load_gather(lc, [i10])
                t = (pw0 * plsc.bitcast(p0, jnp.bfloat16)
                     + pw1 * plsc.bitcast(p1, jnp.bfloat16))
                ta, tb = plsc.unpack(t, format=plsc.PackFormat.INTERLEAVED,
                                     preferred_element_type=jnp.float32)
                obuf[buf, c, pl.ds(s, L)] = ta + tb
        pltpu.make_async_copy(obuf.at[buf], out_hbm.at[b, :, h, :], osem).start()
        return carry

    lax.fori_loop(0, ROWS_PER_W, row_body, 0)
    for tail in (ROWS_PER_W - 2, ROWS_PER_W - 1):
        pltpu.make_async_copy(
            obuf.at[tail % 2], out_hbm.at[b, :, h0 + tail, :], osem).wait()


@jax.jit
def _slice_sc(gt, fm2, tx, xb):
    mesh = plsc.VectorSubcoreMesh(core_axis_name="c", subcore_axis_name="s")
    f = functools.partial(
        pl.kernel,
        out_type=jax.ShapeDtypeStruct((B, C, H, W), jnp.float32),
        mesh=mesh,
        compiler_params=pltpu.CompilerParams(needs_layout_passes=False),
        scratch_types=[
            pltpu.VMEM((C * GH * GW * D + L,), jnp.float32),
            pltpu.VMEM((ROWS_PER_W, W), jnp.float32),
            pltpu.VMEM((W,), jnp.float32),
            pltpu.VMEM((W,), jnp.int32),
            pltpu.VMEM((C * GW * D,), jnp.int32),
            pltpu.VMEM((2, C, W), jnp.float32),
            pltpu.SemaphoreType.DMA,
        ],
    )(_body)
    return f(gt, fm2, tx, xb)


def kernel(bilateral_grid, feature_map):
    gt = jnp.transpose(bilateral_grid, (0, 1, 3, 4, 2)).reshape(B, C * GH * GW * D)
    fm2 = feature_map.reshape(B * H, W)
    return _slice_sc(gt, fm2, jnp.asarray(_TX), jnp.asarray(_XB))


# speedup vs baseline: 1.6394x; 1.0449x over previous
import functools

import numpy as np
import jax
import jax.numpy as jnp
from jax import lax
from jax.experimental import pallas as pl
from jax.experimental.pallas import tpu as pltpu, tpu_sc as plsc

B, C, D, GH, GW = 4, 12, 8, 16, 16
H = W = 512
NC, NS, L = 2, 16, 16
NW = NC * NS
ROWS_PER_W = (B * H) // NW
PV = W // L

_wgf = (np.arange(W, dtype=np.float32) / np.float32(W - 1)) * np.float32(2.0) - np.float32(1.0)
_ix = np.clip((_wgf + np.float32(1.0)) * np.float32(0.5) * np.float32(GW - 1),
              0.0, np.float32(GW - 1)).astype(np.float32)
_x0 = np.minimum(np.floor(_ix).astype(np.int32), GW - 2)
_TX = _ix - _x0.astype(np.float32)
_XB = (_x0 * D).astype(np.int32)


def _body(gt_hbm, fm_hbm, tx_hbm, xb_hbm, out_hbm,
          gridv, fmv, txv, xbv, lv, obuf, osem):
    wid = lax.axis_index("s") * NC + lax.axis_index("c")
    row0 = wid * ROWS_PER_W
    b = row0 // H
    h0 = row0 % H

    pltpu.sync_copy(tx_hbm, txv)
    pltpu.sync_copy(xb_hbm, xbv)
    pltpu.sync_copy(gt_hbm.at[b], gridv)
    pltpu.sync_copy(fm_hbm.at[pl.ds(row0, ROWS_PER_W)], fmv)

    def row_body(r, carry):
        h = h0 + r
        buf = lax.rem(r, 2)
        @pl.when(r >= 2)
        def _():
            pltpu.make_async_copy(obuf.at[buf], out_hbm.at[b, :, h, :], osem).wait()
        hgf = h.astype(jnp.float32) * np.float32(1.0 / (H - 1)) * np.float32(2.0) - np.float32(1.0)
        iy = jnp.clip((hgf + np.float32(1.0)) * np.float32(0.5) * np.float32(GH - 1),
                      np.float32(0.0), np.float32(GH - 1))
        iyv = jnp.full((L,), iy, dtype=jnp.float32)
        y0v = jnp.minimum(iyv.astype(jnp.int32), GH - 2)
        wy1 = iyv - y0v.astype(jnp.float32)
        wy0 = jnp.full((L,), 1.0, dtype=jnp.float32) - wy1
        y0 = jnp.max(y0v)
        ybase = y0 * (GW * D)

        pwy0 = plsc.pack(wy0, wy0, format=plsc.PackFormat.INTERLEAVED)
        pwy1 = plsc.pack(wy1, wy1, format=plsc.PackFormat.INTERLEAVED)

        @plsc.parallel_loop(0, C * (GW * D // L), unroll=8)
        def build(g):
            off = ybase + g * L + lax.shift_right_logical(g, 3) * (GH * GW * D - GW * D)
            gp0 = plsc.bitcast(gridv[pl.ds(off, L)], jnp.bfloat16)
            gp1 = plsc.bitcast(gridv[pl.ds(off + GW * D, L)], jnp.bfloat16)
            t = pwy0 * gp0 + pwy1 * gp1
            lv[pl.ds(g * L, L)] = plsc.bitcast(t, jnp.int32)

        @plsc.parallel_loop(0, PV, unroll=8)
        def pv_body(p):
            s = p * L
            fme = fmv[r, pl.ds(s, L)]
            txe = txv[pl.ds(s, L)]
            xbe = xbv[pl.ds(s, L)]
            iz = jnp.clip((fme + np.float32(1.0)) * np.float32(0.5 * (D - 1)),
                          np.float32(0.0), np.float32(D - 1))
            z0 = jnp.minimum(iz.astype(jnp.int32), D - 2)
            tz = iz - z0.astype(jnp.float32)
            i00 = xbe + z0
            i10 = i00 + D
            one = jnp.full((L,), 1.0, dtype=jnp.float32)
            uz = one - tz
            ux = one - txe
            w00 = ux * uz
            w01 = ux * tz
            w10 = txe * uz
            w11 = txe * tz
            pw0 = plsc.pack(w00, w01, format=plsc.PackFormat.INTERLEAVED)
            pw1 = plsc.pack(w10, w11, format=plsc.PackFormat.INTERLEAVED)
            for c in range(C):
                lc = lv.at[pl.ds(c * (GW * D), GW * D)]
                p0 = plsc.load_gather(lc, [i00])
                p1 = plsc.load_gather(lc, [i10])
                t = (pw0 * plsc.bitcast(p0, jnp.bfloat16)
                     + pw1 * plsc.bitcast(p1, jnp.bfloat16))
                ta, tb = plsc.unpack(t, format=plsc.PackFormat.INTERLEAVED,
                                     preferred_element_type=jnp.float32)
                obuf[buf, c, pl.ds(s, L)] = ta + tb
        pltpu.make_async_copy(obuf.at[buf], out_hbm.at[b, :, h, :], osem).start()
        return carry

    lax.fori_loop(0, ROWS_PER_W, row_body, 0)
    for tail in (ROWS_PER_W - 2, ROWS_PER_W - 1):
        pltpu.make_async_copy(
            obuf.at[tail % 2], out_hbm.at[b, :, h0 + tail, :], osem).wait()


@jax.jit
def _slice_sc(gt, fm2, tx, xb):
    mesh = plsc.VectorSubcoreMesh(core_axis_name="c", subcore_axis_name="s")
    f = functools.partial(
        pl.kernel,
        out_type=jax.ShapeDtypeStruct((B, C, H, W), jnp.float32),
        mesh=mesh,
        compiler_params=pltpu.CompilerParams(needs_layout_passes=False),
        scratch_types=[
            pltpu.VMEM((C * GH * GW * D,), jnp.int32),
            pltpu.VMEM((ROWS_PER_W, W), jnp.float32),
            pltpu.VMEM((W,), jnp.float32),
            pltpu.VMEM((W,), jnp.int32),
            pltpu.VMEM((C * GW * D,), jnp.int32),
            pltpu.VMEM((2, C, W), jnp.float32),
            pltpu.SemaphoreType.DMA,
        ],
    )(_body)
    return f(gt, fm2, tx, xb)


def kernel(bilateral_grid, feature_map):
    gt = jnp.transpose(bilateral_grid, (0, 1, 3, 4, 2))
    nxt = jnp.concatenate([gt[..., 1:], gt[..., -1:]], axis=-1)
    pair = jnp.stack([gt, nxt], axis=-1).astype(jnp.bfloat16)
    gp = jax.lax.bitcast_convert_type(pair, jnp.int32).reshape(B, C * GH * GW * D)
    fm2 = feature_map.reshape(B * H, W)
    return _slice_sc(gp, fm2, jnp.asarray(_TX), jnp.asarray(_XB))


# speedup vs baseline: 1.6452x; 1.0035x over previous
import functools

import numpy as np
import jax
import jax.numpy as jnp
from jax import lax
from jax.experimental import pallas as pl
from jax.experimental.pallas import tpu as pltpu, tpu_sc as plsc

B, C, D, GH, GW = 4, 12, 8, 16, 16
H = W = 512
NC, NS, L = 2, 16, 16
NW = NC * NS
ROWS_PER_W = (B * H) // NW
PV = W // L

_wgf = (np.arange(W, dtype=np.float32) / np.float32(W - 1)) * np.float32(2.0) - np.float32(1.0)
_ix = np.clip((_wgf + np.float32(1.0)) * np.float32(0.5) * np.float32(GW - 1),
              0.0, np.float32(GW - 1)).astype(np.float32)
_x0 = np.minimum(np.floor(_ix).astype(np.int32), GW - 2)
_TX = _ix - _x0.astype(np.float32)
_XB = (_x0 * D).astype(np.int32)


def _body(gt_hbm, fm_hbm, tx_hbm, xb_hbm, out_hbm,
          graw, gridv, fmv, txv, xbv, lv, obuf, osem):
    wid = lax.axis_index("s") * NC + lax.axis_index("c")
    row0 = wid * ROWS_PER_W
    b = row0 // H
    h0 = row0 % H

    pltpu.sync_copy(tx_hbm, txv)
    pltpu.sync_copy(xb_hbm, xbv)
    pltpu.sync_copy(gt_hbm.at[b], graw)
    pltpu.sync_copy(fm_hbm.at[pl.ds(row0, ROWS_PER_W)], fmv)

    lane = lax.iota(jnp.int32, L)
    lx = lax.shift_right_logical(lane, 3)
    ld = lane & 7
    sva = lx + ld * (GH * GW)
    svb = lx + jnp.minimum(ld + 1, D - 1) * (GH * GW)

    @plsc.parallel_loop(0, C * GH * (GW // 2), unroll=8)
    def packgrid(g):
        c = lax.shift_right_logical(g, 7)
        rem = g & 127
        y = lax.shift_right_logical(rem, 3)
        base = c * (D * GH * GW) + y * GW + (rem & 7) * 2
        va = plsc.load_gather(graw, [base + sva])
        vb = plsc.load_gather(graw, [base + svb])
        pk = plsc.pack(va, vb, format=plsc.PackFormat.INTERLEAVED)
        gridv[pl.ds(g * L, L)] = plsc.bitcast(pk, jnp.int32)

    def row_body(r, carry):
        h = h0 + r
        buf = lax.rem(r, 2)
        @pl.when(r >= 2)
        def _():
            pltpu.make_async_copy(obuf.at[buf], out_hbm.at[b, :, h, :], osem).wait()
        hgf = h.astype(jnp.float32) * np.float32(1.0 / (H - 1)) * np.float32(2.0) - np.float32(1.0)
        iy = jnp.clip((hgf + np.float32(1.0)) * np.float32(0.5) * np.float32(GH - 1),
                      np.float32(0.0), np.float32(GH - 1))
        iyv = jnp.full((L,), iy, dtype=jnp.float32)
        y0v = jnp.minimum(iyv.astype(jnp.int32), GH - 2)
        wy1 = iyv - y0v.astype(jnp.float32)
        wy0 = jnp.full((L,), 1.0, dtype=jnp.float32) - wy1
        y0 = jnp.max(y0v)
        ybase = y0 * (GW * D)

        pwy0 = plsc.pack(wy0, wy0, format=plsc.PackFormat.INTERLEAVED)
        pwy1 = plsc.pack(wy1, wy1, format=plsc.PackFormat.INTERLEAVED)

        @plsc.parallel_loop(0, C * (GW * D // L), unroll=8)
        def build(g):
            off = ybase + g * L + lax.shift_right_logical(g, 3) * (GH * GW * D - GW * D)
            gp0 = plsc.bitcast(gridv[pl.ds(off, L)], jnp.bfloat16)
            gp1 = plsc.bitcast(gridv[pl.ds(off + GW * D, L)], jnp.bfloat16)
            t = pwy0 * gp0 + pwy1 * gp1
            lv[pl.ds(g * L, L)] = plsc.bitcast(t, jnp.int32)

        @plsc.parallel_loop(0, PV, unroll=8)
        def pv_body(p):
            s = p * L
            fme = fmv[r, pl.ds(s, L)]
            txe = txv[pl.ds(s, L)]
            xbe = xbv[pl.ds(s, L)]
            iz = jnp.clip((fme + np.float32(1.0)) * np.float32(0.5 * (D - 1)),
                          np.float32(0.0), np.float32(D - 1))
            z0 = jnp.minimum(iz.astype(jnp.int32), D - 2)
            tz = iz - z0.astype(jnp.float32)
            i00 = xbe + z0
            i10 = i00 + D
            one = jnp.full((L,), 1.0, dtype=jnp.float32)
            uz = one - tz
            ux = one - txe
            w00 = ux * uz
            w01 = ux * tz
            w10 = txe * uz
            w11 = txe * tz
            pw0 = plsc.pack(w00, w01, format=plsc.PackFormat.INTERLEAVED)
            pw1 = plsc.pack(w10, w11, format=plsc.PackFormat.INTERLEAVED)
            for c in range(C):
                lc = lv.at[pl.ds(c * (GW * D), GW * D)]
                p0 = plsc.load_gather(lc, [i00])
                p1 = plsc.load_gather(lc, [i10])
                t = (pw0 * plsc.bitcast(p0, jnp.bfloat16)
                     + pw1 * plsc.bitcast(p1, jnp.bfloat16))
                ta, tb = plsc.unpack(t, format=plsc.PackFormat.INTERLEAVED,
                                     preferred_element_type=jnp.float32)
                obuf[buf, c, pl.ds(s, L)] = ta + tb
        pltpu.make_async_copy(obuf.at[buf], out_hbm.at[b, :, h, :], osem).start()
        return carry

    lax.fori_loop(0, ROWS_PER_W, row_body, 0)
    for tail in (ROWS_PER_W - 2, ROWS_PER_W - 1):
        pltpu.make_async_copy(
            obuf.at[tail % 2], out_hbm.at[b, :, h0 + tail, :], osem).wait()


@jax.jit
def _slice_sc(gt, fm2, tx, xb):
    mesh = plsc.VectorSubcoreMesh(core_axis_name="c", subcore_axis_name="s")
    f = functools.partial(
        pl.kernel,
        out_type=jax.ShapeDtypeStruct((B, C, H, W), jnp.float32),
        mesh=mesh,
        compiler_params=pltpu.CompilerParams(needs_layout_passes=False),
        scratch_types=[
            pltpu.VMEM((C * D * GH * GW,), jnp.float32),
            pltpu.VMEM((C * GH * GW * D,), jnp.int32),
            pltpu.VMEM((ROWS_PER_W, W), jnp.float32),
            pltpu.VMEM((W,), jnp.float32),
            pltpu.VMEM((W,), jnp.int32),
            pltpu.VMEM((C * GW * D,), jnp.int32),
            pltpu.VMEM((2, C, W), jnp.float32),
            pltpu.SemaphoreType.DMA,
        ],
    )(_body)
    return f(gt, fm2, tx, xb)


def kernel(bilateral_grid, feature_map):
    graw = bilateral_grid.reshape(B, C * D * GH * GW)
    fm2 = feature_map.reshape(B * H, W)
    return _slice_sc(graw, fm2, jnp.asarray(_TX), jnp.asarray(_XB))
